# Initial kernel scaffold; baseline (speedup 1.0000x reference)
import jax
import jax.numpy as jnp
from jax.experimental import pallas as pl


def kernel(y_pre, y_batch):
    raise NotImplementedError("write your pallas kernel here")


# speedup vs baseline: 7.8928x; 7.8928x over previous
import functools

import jax
import jax.numpy as jnp
from jax.experimental import pallas as pl

_N = 20000
_B = 64
_NUM_CLASSES = 7
_BETA = 0.5
_IMIN = -2147483648
_MASK = 0x7FFFFFFF


def _phase1_kernel(yp_ref, yb_ref, keys_ref, ploss_ref, npos_ref, box_ref):
    cp = yp_ref[0]
    ch = yb_ref[0]
    cp7 = cp[0:_NUM_CLASSES, :]
    ch7 = ch[0:_NUM_CLASSES, :]

    m_hat = jnp.max(ch7, axis=0, keepdims=True)
    pos = (m_hat > 0.0) & (ch[0:1, :] < m_hat)
    iota7 = jax.lax.broadcasted_iota(jnp.int32, (_NUM_CLASSES, _N), 0)
    eq = ch7 == m_hat
    first_idx = jnp.min(jnp.where(eq, iota7, _NUM_CLASSES), axis=0, keepdims=True)
    ff = iota7 == first_idx

    m_pre = jnp.max(cp7, axis=0, keepdims=True)
    se = jnp.sum(jnp.exp(cp7 - m_pre), axis=0, keepdims=True)
    lse = m_pre + jnp.log(se)
    cpt = jnp.sum(jnp.where(ff, cp7, 0.0), axis=0, keepdims=True)
    ce = lse - cpt
    ce_bg = lse - cp[0:1, :]

    d = cp[_NUM_CLASSES:, :] - ch[_NUM_CLASSES:, :]
    ad = jnp.abs(d)
    sl1 = jnp.where(ad < 1.0, 0.5 * d * d, ad - 0.5)
    box_row = jnp.sum(jnp.where(pos, jnp.sum(sl1, axis=0, keepdims=True), 0.0))

    num_pos = jnp.sum(pos.astype(jnp.int32))
    pos_loss = jnp.sum(jnp.where(pos, ce, 0.0))

    bits = jax.lax.bitcast_convert_type(ce_bg, jnp.int32)
    key = jnp.where(bits >= 0, bits, bits ^ _MASK)
    key = jnp.where(pos, _IMIN, key)

    keys_ref[0] = key
    ploss_ref[...] = jnp.reshape(pos_loss, (1, 1, 1))
    npos_ref[...] = jnp.reshape(num_pos, (1, 1, 1))
    box_ref[...] = jnp.reshape(box_row, (1, 1, 1))


def _phase2_kernel(keys_ref, ploss_ref, npos_ref, box_ref,
                   total_ref, lclass_ref, lbox_ref):
    u = keys_ref[...]
    npos = npos_ref[...]
    ploss = ploss_ref[...]
    nneg = _N - npos
    k = jnp.minimum(3 * npos, nneg)

    cnt0 = jnp.sum((u >= 0).astype(jnp.int32), axis=1, keepdims=True)
    thresh0 = jnp.where(cnt0 >= k, jnp.int32(0), _IMIN)

    def body(i, t):
        bit = jnp.int32(1) << (30 - i)
        cand = t + bit
        cnt = jnp.sum((u >= cand).astype(jnp.int32), axis=1, keepdims=True)
        return jnp.where(cnt >= k, cand, t)

    t_key = jax.lax.fori_loop(0, 31, body, thresh0)

    gt = u > t_key
    cnt_gt = jnp.sum(gt.astype(jnp.int32), axis=1, keepdims=True)
    vi = jnp.where(u >= 0, u, u ^ _MASK)
    v = jax.lax.bitcast_convert_type(vi, jnp.float32)
    sum_gt = jnp.sum(jnp.where(gt, v, 0.0), axis=1, keepdims=True)
    ti = jnp.where(t_key >= 0, t_key, t_key ^ _MASK)
    tval = jax.lax.bitcast_convert_type(ti, jnp.float32)
    neg_loss = jnp.where(k > 0,
                         sum_gt + (k - cnt_gt).astype(jnp.float32) * tval,
                         0.0)

    npf = npos.astype(jnp.float32)
    denom = (npos + k).astype(jnp.float32)
    l_i = jnp.where(nneg > 0,
                    (ploss + neg_loss) / jnp.maximum(denom, 1.0),
                    ploss / jnp.maximum(npf, 1.0))
    has_pos = npos > 0
    n_valid = jnp.sum(has_pos.astype(jnp.int32))
    sum_li = jnp.sum(jnp.where(has_pos, l_i, 0.0))
    l_class = jnp.where(n_valid > 0,
                        sum_li / jnp.maximum(n_valid, 1).astype(jnp.float32),
                        0.0)
    total_pos = jnp.sum(npos)
    box_total = jnp.sum(box_ref[...])
    l_box = jnp.where(total_pos > 0,
                      box_total / (total_pos.astype(jnp.float32) + 1e-6),
                      0.0)
    total_ref[...] = jnp.reshape(l_class + _BETA * l_box, (1, 1))
    lclass_ref[...] = jnp.reshape(l_class, (1, 1))
    lbox_ref[...] = jnp.reshape(l_box, (1, 1))


@jax.jit
def kernel(y_pre, y_batch):
    yp = jnp.transpose(y_pre, (0, 2, 1))
    yb = jnp.transpose(y_batch, (0, 2, 1))

    keys, ploss, npos, box = pl.pallas_call(
        _phase1_kernel,
        grid=(_B,),
        in_specs=[
            pl.BlockSpec((1, 11, _N), lambda i: (i, 0, 0)),
            pl.BlockSpec((1, 11, _N), lambda i: (i, 0, 0)),
        ],
        out_specs=[
            pl.BlockSpec((1, 1, _N), lambda i: (i, 0, 0)),
            pl.BlockSpec((1, 1, 1), lambda i: (i, 0, 0)),
            pl.BlockSpec((1, 1, 1), lambda i: (i, 0, 0)),
            pl.BlockSpec((1, 1, 1), lambda i: (i, 0, 0)),
        ],
        out_shape=[
            jax.ShapeDtypeStruct((_B, 1, _N), jnp.int32),
            jax.ShapeDtypeStruct((_B, 1, 1), jnp.float32),
            jax.ShapeDtypeStruct((_B, 1, 1), jnp.int32),
            jax.ShapeDtypeStruct((_B, 1, 1), jnp.float32),
        ],
    )(yp, yb)

    total, l_class, l_box = pl.pallas_call(
        _phase2_kernel,
        out_shape=[
            jax.ShapeDtypeStruct((1, 1), jnp.float32),
            jax.ShapeDtypeStruct((1, 1), jnp.float32),
            jax.ShapeDtypeStruct((1, 1), jnp.float32),
        ],
    )(keys.reshape(_B, _N), ploss.reshape(_B, 1),
      npos.reshape(_B, 1), box.reshape(_B, 1))

    return (total[0, 0], l_class[0, 0], l_box[0, 0])
